# Initial kernel scaffold; baseline (speedup 1.0000x reference)
import jax
import jax.numpy as jnp
from jax.experimental import pallas as pl


def kernel(x, edge_index, W1l, b1, W1r, W2l, b2, W2r, W3l, b3, W3r, W4l, b4, W4r):
    raise NotImplementedError("write your pallas kernel here")


# speedup vs baseline: 6.1005x; 6.1005x over previous
import functools

import jax
import jax.numpy as jnp
from jax import lax
from jax.experimental import pallas as pl
from jax.experimental.pallas import tpu as pltpu
from jax.experimental.pallas import tpu_sc as plsc

N = 10000
E = 320000
NC = 2
NS = 16
NW = NC * NS
EPW = E // NW
CH = 80
NCHUNK = EPW // CH
NPAD = 10240
RPT = NPAD // NS
CNTW = 8


def _seg_sum_sc(table, src, dst, with_cnt):
    C = table.shape[1]
    mesh = plsc.VectorSubcoreMesh(core_axis_name="c", subcore_axis_name="s")
    out_type = [jax.ShapeDtypeStruct((NC, NPAD, C), jnp.float32)]
    scratch = [
        pltpu.VMEM((CH,), jnp.int32),
        pltpu.VMEM((CH,), jnp.int32),
        pltpu.VMEM((CH, C), jnp.float32),
        pltpu.VMEM_SHARED((NPAD, C), jnp.float32),
        pltpu.SemaphoreType.DMA,
    ]
    if with_cnt:
        out_type.append(jax.ShapeDtypeStruct((NC, NPAD, CNTW), jnp.float32))
        scratch += [
            pltpu.VMEM((CH, CNTW), jnp.float32),
            pltpu.VMEM_SHARED((NPAD, CNTW), jnp.float32),
        ]

    zeros_c = jnp.zeros((NPAD, C), jnp.float32)
    if with_cnt:
        zeros_w = jnp.zeros((NPAD, CNTW), jnp.float32)
        ones_w = jnp.ones((CH, CNTW), jnp.float32)

        def body(tbl, src_h, dst_h, zc_h, zw_h, ones_h, acc_o, cnt_o,
                 sidx, didx, rows, acc_sh, sem, ones_v, cnt_sh):
            c = lax.axis_index("c")
            s = lax.axis_index("s")
            r0 = s * RPT
            pltpu.sync_copy(zc_h.at[pl.ds(r0, RPT)], acc_sh.at[pl.ds(r0, RPT)])
            pltpu.sync_copy(zw_h.at[pl.ds(r0, RPT)], cnt_sh.at[pl.ds(r0, RPT)])
            pltpu.sync_copy(ones_h, ones_v)
            plsc.subcore_barrier()
            base = (c * NS + s) * EPW

            def chunk(j, carry):
                off = base + j * CH
                pltpu.sync_copy(src_h.at[pl.ds(off, CH)], sidx)
                pltpu.sync_copy(dst_h.at[pl.ds(off, CH)], didx)
                pltpu.async_copy(tbl.at[sidx], rows, sem).wait()
                pltpu.sync_copy(rows, acc_sh.at[didx], add=True)
                pltpu.sync_copy(ones_v, cnt_sh.at[didx], add=True)
                return carry

            lax.fori_loop(0, NCHUNK, chunk, 0)
            plsc.subcore_barrier()
            pltpu.sync_copy(acc_sh.at[pl.ds(r0, RPT)],
                            acc_o.at[c, pl.ds(r0, RPT)])
            pltpu.sync_copy(cnt_sh.at[pl.ds(r0, RPT)],
                            cnt_o.at[c, pl.ds(r0, RPT)])

        k = pl.kernel(body, out_type=out_type, mesh=mesh,
                      scratch_types=scratch,
                      compiler_params=pltpu.CompilerParams(
                          use_tc_tiling_on_sc=False))
        return k(table, src, dst, zeros_c, zeros_w, ones_w)

    def body(tbl, src_h, dst_h, zc_h, acc_o, sidx, didx, rows, acc_sh, sem):
        c = lax.axis_index("c")
        s = lax.axis_index("s")
        r0 = s * RPT
        pltpu.sync_copy(zc_h.at[pl.ds(r0, RPT)], acc_sh.at[pl.ds(r0, RPT)])
        plsc.subcore_barrier()
        base = (c * NS + s) * EPW

        def chunk(j, carry):
            off = base + j * CH
            pltpu.sync_copy(src_h.at[pl.ds(off, CH)], sidx)
            pltpu.sync_copy(dst_h.at[pl.ds(off, CH)], didx)
            pltpu.async_copy(tbl.at[sidx], rows, sem).wait()
            pltpu.sync_copy(rows, acc_sh.at[didx], add=True)
            return carry

        lax.fori_loop(0, NCHUNK, chunk, 0)
        plsc.subcore_barrier()
        pltpu.sync_copy(acc_sh.at[pl.ds(r0, RPT)], acc_o.at[c, pl.ds(r0, RPT)])

    k = pl.kernel(body, out_type=out_type[0], mesh=mesh,
                  scratch_types=scratch,
                  compiler_params=pltpu.CompilerParams(
                      use_tc_tiling_on_sc=False))
    return k(table, src, dst, zeros_c)


def _tc_pre(x, W3l, W3r):
    BN = 1000
    D = x.shape[1]
    H = W3l.shape[1]

    def body(x_ref, wl_ref, wr_ref, p_ref, xr_ref):
        xb = x_ref[...]
        p_ref[...] = jnp.dot(xb, wl_ref[...], preferred_element_type=jnp.float32)
        xr_ref[...] = jnp.dot(xb, wr_ref[...], preferred_element_type=jnp.float32)

    return pl.pallas_call(
        body,
        grid=(N // BN,),
        in_specs=[
            pl.BlockSpec((BN, D), lambda i: (i, 0)),
            pl.BlockSpec((D, H), lambda i: (0, 0)),
            pl.BlockSpec((D, H), lambda i: (0, 0)),
        ],
        out_specs=[pl.BlockSpec((BN, H), lambda i: (i, 0))] * 2,
        out_shape=[jax.ShapeDtypeStruct((N, H), jnp.float32)] * 2,
    )(x, W3l, W3r)


def _tc_mid(acc, cnt, xr, b3):
    BN = 1000
    H = xr.shape[1]

    def body(a_ref, c_ref, xr_ref, b_ref, h_ref):
        a = a_ref[...]
        cn = c_ref[...]
        ssum = a[0] + a[1]
        deg = cn[0, :, :1] + cn[1, :, :1]
        h_ref[...] = jnp.maximum(
            ssum / jnp.maximum(deg, 1.0) + b_ref[...] + xr_ref[...], 0.0)

    return pl.pallas_call(
        body,
        grid=(N // BN,),
        in_specs=[
            pl.BlockSpec((NC, BN, H), lambda i: (0, i, 0)),
            pl.BlockSpec((NC, BN, CNTW), lambda i: (0, i, 0)),
            pl.BlockSpec((BN, H), lambda i: (i, 0)),
            pl.BlockSpec((1, H), lambda i: (0, 0)),
        ],
        out_specs=pl.BlockSpec((BN, H), lambda i: (i, 0)),
        out_shape=jax.ShapeDtypeStruct((N, H), jnp.float32),
    )(acc, cnt, xr, b3)


def _tc_out(acc, cnt, h, W4l, W4r, b4):
    BN = 1000
    H = h.shape[1]
    O = W4l.shape[1]

    def body(a_ref, c_ref, h_ref, wl_ref, wr_ref, b_ref, o_ref):
        a = a_ref[...]
        cn = c_ref[...]
        deg = cn[0, :, :1] + cn[1, :, :1]
        mean = (a[0] + a[1]) / jnp.maximum(deg, 1.0)
        o = (jnp.dot(mean, wl_ref[...], preferred_element_type=jnp.float32)
             + b_ref[...]
             + jnp.dot(h_ref[...], wr_ref[...],
                       preferred_element_type=jnp.float32))
        m = jnp.max(o, axis=1, keepdims=True)
        eo = jnp.exp(o - m)
        o_ref[...] = o - m - jnp.log(jnp.sum(eo, axis=1, keepdims=True))

    return pl.pallas_call(
        body,
        grid=(N // BN,),
        in_specs=[
            pl.BlockSpec((NC, BN, H), lambda i: (0, i, 0)),
            pl.BlockSpec((NC, BN, CNTW), lambda i: (0, i, 0)),
            pl.BlockSpec((BN, H), lambda i: (i, 0)),
            pl.BlockSpec((H, O), lambda i: (0, 0)),
            pl.BlockSpec((H, O), lambda i: (0, 0)),
            pl.BlockSpec((1, O), lambda i: (0, 0)),
        ],
        out_specs=pl.BlockSpec((BN, O), lambda i: (i, 0)),
        out_shape=jax.ShapeDtypeStruct((N, O), jnp.float32),
    )(acc, cnt, h, W4l, W4r, b4)


def kernel(x, edge_index, W1l, b1, W1r, W2l, b2, W2r, W3l, b3, W3r,
           W4l, b4, W4r):
    src = edge_index[0]
    dst = edge_index[1]
    p, xr = _tc_pre(x, W3l, W3r)
    acc1, cnt = _seg_sum_sc(p, src, dst, with_cnt=True)
    h = _tc_mid(acc1, cnt, xr, b3.reshape(1, -1))
    acc2 = _seg_sum_sc(h, src, dst, with_cnt=False)
    return _tc_out(acc2, cnt, h, W4l, W4r, b4.reshape(1, -1))


# speedup vs baseline: 15.2336x; 2.4971x over previous
import functools

import jax
import jax.numpy as jnp
from jax import lax
from jax.experimental import pallas as pl
from jax.experimental.pallas import tpu as pltpu
from jax.experimental.pallas import tpu_sc as plsc

N = 10000
E = 320000
NC = 2
NS = 16
NW = NC * NS
EPW = E // NW
CH = 125
NCHUNK = EPW // CH
NBUF = 4
NPAD = 10240
RPT = NPAD // NS
CNTW = 8


def _seg_sum_sc(table, src3, dst3, with_cnt):
    C = table.shape[1]
    mesh = plsc.VectorSubcoreMesh(core_axis_name="c", subcore_axis_name="s")
    out_type = [jax.ShapeDtypeStruct((NC, NPAD, C), jnp.float32)]
    scratch = [
        pltpu.VMEM((NCHUNK, CH), jnp.int32),
        pltpu.VMEM((NCHUNK, CH), jnp.int32),
        [pltpu.VMEM((CH, C), jnp.float32) for _ in range(NBUF)],
        [pltpu.SemaphoreType.DMA for _ in range(NBUF)],
        pltpu.VMEM_SHARED((NPAD, C), jnp.float32),
    ]
    if with_cnt:
        out_type.append(jax.ShapeDtypeStruct((NC, NPAD, CNTW), jnp.float32))
        scratch += [
            pltpu.VMEM((CH, CNTW), jnp.float32),
            pltpu.VMEM_SHARED((NPAD, CNTW), jnp.float32),
        ]

    zeros_c = jnp.zeros((NPAD, C), jnp.float32)
    if with_cnt:
        zeros_w = jnp.zeros((NPAD, CNTW), jnp.float32)
        ones_w = jnp.ones((CH, CNTW), jnp.float32)

    def body(*refs):
        if with_cnt:
            (tbl, src_h, dst_h, zc_h, zw_h, ones_h, acc_o, cnt_o,
             sidx, didx, rows, sems, acc_sh, ones_v, cnt_sh) = refs
        else:
            (tbl, src_h, dst_h, zc_h, acc_o,
             sidx, didx, rows, sems, acc_sh) = refs
        c = lax.axis_index("c")
        s = lax.axis_index("s")
        wid = c * NS + s
        r0 = s * RPT
        pltpu.sync_copy(zc_h.at[pl.ds(r0, RPT)], acc_sh.at[pl.ds(r0, RPT)])
        pltpu.sync_copy(src_h.at[wid], sidx)
        pltpu.sync_copy(dst_h.at[wid], didx)
        if with_cnt:
            pltpu.sync_copy(zw_h.at[pl.ds(r0, RPT)], cnt_sh.at[pl.ds(r0, RPT)])
            pltpu.sync_copy(ones_h, ones_v)
        plsc.subcore_barrier()

        @pl.loop(0, NCHUNK, step=NBUF)
        def group(g):
            ds = [pltpu.async_copy(tbl.at[sidx.at[g + b]], rows[b], sems[b])
                  for b in range(NBUF)]
            for b in range(NBUF):
                ds[b].wait()
                pltpu.sync_copy(rows[b], acc_sh.at[didx.at[g + b]], add=True)
                if with_cnt:
                    pltpu.sync_copy(ones_v, cnt_sh.at[didx.at[g + b]],
                                    add=True)

        plsc.subcore_barrier()
        pltpu.sync_copy(acc_sh.at[pl.ds(r0, RPT)], acc_o.at[c, pl.ds(r0, RPT)])
        if with_cnt:
            pltpu.sync_copy(cnt_sh.at[pl.ds(r0, RPT)],
                            cnt_o.at[c, pl.ds(r0, RPT)])

    params = pltpu.CompilerParams(use_tc_tiling_on_sc=False)
    if with_cnt:
        k = pl.kernel(body, out_type=out_type, mesh=mesh,
                      scratch_types=scratch, compiler_params=params)
        return k(table, src3, dst3, zeros_c, zeros_w, ones_w)
    k = pl.kernel(body, out_type=out_type[0], mesh=mesh,
                  scratch_types=scratch, compiler_params=params)
    return k(table, src3, dst3, zeros_c)


def _tc_pre(x, W3l, W3r):
    BN = 1000
    D = x.shape[1]
    H = W3l.shape[1]

    def body(x_ref, wl_ref, wr_ref, p_ref, xr_ref):
        xb = x_ref[...]
        p_ref[...] = jnp.dot(xb, wl_ref[...], preferred_element_type=jnp.float32)
        xr_ref[...] = jnp.dot(xb, wr_ref[...], preferred_element_type=jnp.float32)

    return pl.pallas_call(
        body,
        grid=(N // BN,),
        in_specs=[
            pl.BlockSpec((BN, D), lambda i: (i, 0)),
            pl.BlockSpec((D, H), lambda i: (0, 0)),
            pl.BlockSpec((D, H), lambda i: (0, 0)),
        ],
        out_specs=[pl.BlockSpec((BN, H), lambda i: (i, 0))] * 2,
        out_shape=[jax.ShapeDtypeStruct((N, H), jnp.float32)] * 2,
    )(x, W3l, W3r)


def _tc_mid(acc, cnt, xr, b3):
    BN = 1000
    H = xr.shape[1]

    def body(a_ref, c_ref, xr_ref, b_ref, h_ref):
        a = a_ref[...]
        cn = c_ref[...]
        ssum = a[0] + a[1]
        deg = cn[0, :, :1] + cn[1, :, :1]
        h_ref[...] = jnp.maximum(
            ssum / jnp.maximum(deg, 1.0) + b_ref[...] + xr_ref[...], 0.0)

    return pl.pallas_call(
        body,
        grid=(N // BN,),
        in_specs=[
            pl.BlockSpec((NC, BN, H), lambda i: (0, i, 0)),
            pl.BlockSpec((NC, BN, CNTW), lambda i: (0, i, 0)),
            pl.BlockSpec((BN, H), lambda i: (i, 0)),
            pl.BlockSpec((1, H), lambda i: (0, 0)),
        ],
        out_specs=pl.BlockSpec((BN, H), lambda i: (i, 0)),
        out_shape=jax.ShapeDtypeStruct((N, H), jnp.float32),
    )(acc, cnt, xr, b3)


def _tc_out(acc, cnt, h, W4l, W4r, b4):
    BN = 1000
    H = h.shape[1]
    O = W4l.shape[1]

    def body(a_ref, c_ref, h_ref, wl_ref, wr_ref, b_ref, o_ref):
        a = a_ref[...]
        cn = c_ref[...]
        deg = cn[0, :, :1] + cn[1, :, :1]
        mean = (a[0] + a[1]) / jnp.maximum(deg, 1.0)
        o = (jnp.dot(mean, wl_ref[...], preferred_element_type=jnp.float32)
             + b_ref[...]
             + jnp.dot(h_ref[...], wr_ref[...],
                       preferred_element_type=jnp.float32))
        m = jnp.max(o, axis=1, keepdims=True)
        eo = jnp.exp(o - m)
        o_ref[...] = o - m - jnp.log(jnp.sum(eo, axis=1, keepdims=True))

    return pl.pallas_call(
        body,
        grid=(N // BN,),
        in_specs=[
            pl.BlockSpec((NC, BN, H), lambda i: (0, i, 0)),
            pl.BlockSpec((NC, BN, CNTW), lambda i: (0, i, 0)),
            pl.BlockSpec((BN, H), lambda i: (i, 0)),
            pl.BlockSpec((H, O), lambda i: (0, 0)),
            pl.BlockSpec((H, O), lambda i: (0, 0)),
            pl.BlockSpec((1, O), lambda i: (0, 0)),
        ],
        out_specs=pl.BlockSpec((BN, O), lambda i: (i, 0)),
        out_shape=jax.ShapeDtypeStruct((N, O), jnp.float32),
    )(acc, cnt, h, W4l, W4r, b4)


def kernel(x, edge_index, W1l, b1, W1r, W2l, b2, W2r, W3l, b3, W3r,
           W4l, b4, W4r):
    src3 = edge_index[0].reshape(NW, NCHUNK, CH)
    dst3 = edge_index[1].reshape(NW, NCHUNK, CH)
    p, xr = _tc_pre(x, W3l, W3r)
    acc1, cnt = _seg_sum_sc(p, src3, dst3, with_cnt=True)
    h = _tc_mid(acc1, cnt, xr, b3.reshape(1, -1))
    acc2 = _seg_sum_sc(h, src3, dst3, with_cnt=False)
    return _tc_out(acc2, cnt, h, W4l, W4r, b4.reshape(1, -1))


# speedup vs baseline: 16.0402x; 1.0530x over previous
import functools

import jax
import jax.numpy as jnp
from jax import lax
from jax.experimental import pallas as pl
from jax.experimental.pallas import tpu as pltpu
from jax.experimental.pallas import tpu_sc as plsc

N = 10000
E = 320000
NC = 2
NS = 16
NW = NC * NS
EPW = E // NW
CH = 125
NCHUNK = EPW // CH
NBUF = 4
NPAD = 10240
RPT = NPAD // NS
CNTW = 8


def _seg_sum_sc(table, edge3, with_cnt):
    C = table.shape[1]
    mesh = plsc.VectorSubcoreMesh(core_axis_name="c", subcore_axis_name="s")
    out_type = [jax.ShapeDtypeStruct((NC, NPAD, C), jnp.float32)]
    scratch = [
        pltpu.VMEM((NCHUNK, CH), jnp.int32),
        pltpu.VMEM((NCHUNK, CH), jnp.int32),
        [pltpu.VMEM((CH, C), jnp.float32) for _ in range(NBUF)],
        [pltpu.SemaphoreType.DMA for _ in range(NBUF)],
        pltpu.VMEM_SHARED((NPAD, C), jnp.float32),
    ]
    if with_cnt:
        out_type.append(jax.ShapeDtypeStruct((NC, NPAD, CNTW), jnp.float32))
        scratch += [
            pltpu.VMEM((CH, CNTW), jnp.float32),
            pltpu.VMEM_SHARED((NPAD, CNTW), jnp.float32),
        ]

    zeros_c = jnp.zeros((NPAD, C), jnp.float32)
    if with_cnt:
        zeros_w = jnp.zeros((NPAD, CNTW), jnp.float32)
        ones_w = jnp.ones((CH, CNTW), jnp.float32)

    def body(*refs):
        if with_cnt:
            (tbl, e_h, zc_h, zw_h, ones_h, acc_o, cnt_o,
             sidx, didx, rows, sems, acc_sh, ones_v, cnt_sh) = refs
        else:
            (tbl, e_h, zc_h, acc_o,
             sidx, didx, rows, sems, acc_sh) = refs
        c = lax.axis_index("c")
        s = lax.axis_index("s")
        wid = c * NS + s
        r0 = s * RPT
        pltpu.sync_copy(zc_h.at[pl.ds(r0, RPT)], acc_sh.at[pl.ds(r0, RPT)])
        pltpu.sync_copy(e_h.at[0, wid], sidx)
        pltpu.sync_copy(e_h.at[1, wid], didx)
        if with_cnt:
            pltpu.sync_copy(zw_h.at[pl.ds(r0, RPT)], cnt_sh.at[pl.ds(r0, RPT)])
            pltpu.sync_copy(ones_h, ones_v)
        plsc.subcore_barrier()

        @pl.loop(0, NCHUNK, step=NBUF)
        def group(g):
            ds = [pltpu.async_copy(tbl.at[sidx.at[g + b]], rows[b], sems[b])
                  for b in range(NBUF)]
            for b in range(NBUF):
                ds[b].wait()
                pltpu.sync_copy(rows[b], acc_sh.at[didx.at[g + b]], add=True)
                if with_cnt:
                    pltpu.sync_copy(ones_v, cnt_sh.at[didx.at[g + b]],
                                    add=True)

        plsc.subcore_barrier()
        pltpu.sync_copy(acc_sh.at[pl.ds(r0, RPT)], acc_o.at[c, pl.ds(r0, RPT)])
        if with_cnt:
            pltpu.sync_copy(cnt_sh.at[pl.ds(r0, RPT)],
                            cnt_o.at[c, pl.ds(r0, RPT)])

    params = pltpu.CompilerParams(use_tc_tiling_on_sc=False)
    if with_cnt:
        k = pl.kernel(body, out_type=out_type, mesh=mesh,
                      scratch_types=scratch, compiler_params=params)
        return k(table, edge3, zeros_c, zeros_w, ones_w)
    k = pl.kernel(body, out_type=out_type[0], mesh=mesh,
                  scratch_types=scratch, compiler_params=params)
    return k(table, edge3, zeros_c)


def _tc_pre(x, W3l, W3r):
    BN = 1000
    D = x.shape[1]
    H = W3l.shape[1]

    def body(x_ref, wl_ref, wr_ref, p_ref, xr_ref):
        xb = x_ref[...]
        p_ref[...] = jnp.dot(xb, wl_ref[...], preferred_element_type=jnp.float32)
        xr_ref[...] = jnp.dot(xb, wr_ref[...], preferred_element_type=jnp.float32)

    return pl.pallas_call(
        body,
        grid=(N // BN,),
        in_specs=[
            pl.BlockSpec((BN, D), lambda i: (i, 0)),
            pl.BlockSpec((D, H), lambda i: (0, 0)),
            pl.BlockSpec((D, H), lambda i: (0, 0)),
        ],
        out_specs=[pl.BlockSpec((BN, H), lambda i: (i, 0))] * 2,
        out_shape=[jax.ShapeDtypeStruct((N, H), jnp.float32)] * 2,
    )(x, W3l, W3r)


def _tc_mid(acc, cnt, xr, b3):
    BN = 1000
    H = xr.shape[1]

    def body(a_ref, c_ref, xr_ref, b_ref, h_ref):
        a = a_ref[...]
        cn = c_ref[...]
        ssum = a[0] + a[1]
        deg = cn[0, :, :1] + cn[1, :, :1]
        h_ref[...] = jnp.maximum(
            ssum / jnp.maximum(deg, 1.0) + b_ref[...] + xr_ref[...], 0.0)

    return pl.pallas_call(
        body,
        grid=(N // BN,),
        in_specs=[
            pl.BlockSpec((NC, BN, H), lambda i: (0, i, 0)),
            pl.BlockSpec((NC, BN, CNTW), lambda i: (0, i, 0)),
            pl.BlockSpec((BN, H), lambda i: (i, 0)),
            pl.BlockSpec((1, H), lambda i: (0, 0)),
        ],
        out_specs=pl.BlockSpec((BN, H), lambda i: (i, 0)),
        out_shape=jax.ShapeDtypeStruct((N, H), jnp.float32),
    )(acc, cnt, xr, b3)


def _tc_out(acc, cnt, h, W4l, W4r, b4):
    BN = 1000
    H = h.shape[1]
    O = W4l.shape[1]

    def body(a_ref, c_ref, h_ref, wl_ref, wr_ref, b_ref, o_ref):
        a = a_ref[...]
        cn = c_ref[...]
        deg = cn[0, :, :1] + cn[1, :, :1]
        mean = (a[0] + a[1]) / jnp.maximum(deg, 1.0)
        o = (jnp.dot(mean, wl_ref[...], preferred_element_type=jnp.float32)
             + b_ref[...]
             + jnp.dot(h_ref[...], wr_ref[...],
                       preferred_element_type=jnp.float32))
        m = jnp.max(o, axis=1, keepdims=True)
        eo = jnp.exp(o - m)
        o_ref[...] = o - m - jnp.log(jnp.sum(eo, axis=1, keepdims=True))

    return pl.pallas_call(
        body,
        grid=(N // BN,),
        in_specs=[
            pl.BlockSpec((NC, BN, H), lambda i: (0, i, 0)),
            pl.BlockSpec((NC, BN, CNTW), lambda i: (0, i, 0)),
            pl.BlockSpec((BN, H), lambda i: (i, 0)),
            pl.BlockSpec((H, O), lambda i: (0, 0)),
            pl.BlockSpec((H, O), lambda i: (0, 0)),
            pl.BlockSpec((1, O), lambda i: (0, 0)),
        ],
        out_specs=pl.BlockSpec((BN, O), lambda i: (i, 0)),
        out_shape=jax.ShapeDtypeStruct((N, O), jnp.float32),
    )(acc, cnt, h, W4l, W4r, b4)


def kernel(x, edge_index, W1l, b1, W1r, W2l, b2, W2r, W3l, b3, W3r,
           W4l, b4, W4r):
    edge3 = edge_index.reshape(2, NW, NCHUNK, CH)
    p, xr = _tc_pre(x, W3l, W3r)
    acc1, cnt = _seg_sum_sc(p, edge3, with_cnt=True)
    h = _tc_mid(acc1, cnt, xr, b3.reshape(1, -1))
    acc2 = _seg_sum_sc(h, edge3, with_cnt=False)
    return _tc_out(acc2, cnt, h, W4l, W4r, b4.reshape(1, -1))


# speedup vs baseline: 16.2964x; 1.0160x over previous
import functools

import jax
import jax.numpy as jnp
from jax import lax
from jax.experimental import pallas as pl
from jax.experimental.pallas import tpu as pltpu
from jax.experimental.pallas import tpu_sc as plsc

N = 10000
E = 320000
NC = 2
NS = 16
NW = NC * NS
EPW = E // NW
CH = 125
NCHUNK = EPW // CH
NBUF = 4
NPAD = 10240
RPT = NPAD // NS
CNTW = 8


def _seg_sum_sc(table, edge3, with_cnt):
    C = table.shape[1]
    mesh = plsc.VectorSubcoreMesh(core_axis_name="c", subcore_axis_name="s")
    out_type = [jax.ShapeDtypeStruct((NC, NPAD, C), jnp.float32)]
    scratch = [
        pltpu.VMEM((NCHUNK, CH), jnp.int32),
        pltpu.VMEM((NCHUNK, CH), jnp.int32),
        [pltpu.VMEM((CH, C), jnp.float32) for _ in range(NBUF)],
        [pltpu.SemaphoreType.DMA for _ in range(NBUF)],
        pltpu.VMEM_SHARED((NPAD, C), jnp.float32),
    ]
    if with_cnt:
        out_type.append(jax.ShapeDtypeStruct((NC, NPAD, CNTW), jnp.float32))
        scratch += [
            pltpu.VMEM((CH, CNTW), jnp.float32),
            pltpu.VMEM_SHARED((NPAD, CNTW), jnp.float32),
        ]

    zeros_c = jnp.zeros((NPAD, C), jnp.float32)
    if with_cnt:
        zeros_w = jnp.zeros((NPAD, CNTW), jnp.float32)
        ones_w = jnp.ones((CH, CNTW), jnp.float32)

    def body(*refs):
        if with_cnt:
            (tbl, e_h, zc_h, zw_h, ones_h, acc_o, cnt_o,
             sidx, didx, rows, sems, acc_sh, ones_v, cnt_sh) = refs
        else:
            (tbl, e_h, zc_h, acc_o,
             sidx, didx, rows, sems, acc_sh) = refs
        c = lax.axis_index("c")
        s = lax.axis_index("s")
        wid = c * NS + s
        r0 = s * RPT
        pltpu.sync_copy(zc_h.at[pl.ds(r0, RPT)], acc_sh.at[pl.ds(r0, RPT)])
        pltpu.sync_copy(e_h.at[0, wid], sidx)
        pltpu.sync_copy(e_h.at[1, wid], didx)
        if with_cnt:
            pltpu.sync_copy(zw_h.at[pl.ds(r0, RPT)], cnt_sh.at[pl.ds(r0, RPT)])
            pltpu.sync_copy(ones_h, ones_v)
        plsc.subcore_barrier()

        @pl.loop(0, NCHUNK, step=NBUF)
        def group(g):
            ds = [pltpu.async_copy(tbl.at[sidx.at[g + b]], rows[b], sems[b])
                  for b in range(NBUF)]
            for b in range(NBUF):
                ds[b].wait()
                pltpu.sync_copy(rows[b], acc_sh.at[didx.at[g + b]], add=True)
                if with_cnt:
                    pltpu.sync_copy(ones_v, cnt_sh.at[didx.at[g + b]],
                                    add=True)

        plsc.subcore_barrier()
        pltpu.sync_copy(acc_sh.at[pl.ds(r0, RPT)], acc_o.at[c, pl.ds(r0, RPT)])
        if with_cnt:
            pltpu.sync_copy(cnt_sh.at[pl.ds(r0, RPT)],
                            cnt_o.at[c, pl.ds(r0, RPT)])

    params = pltpu.CompilerParams(use_tc_tiling_on_sc=False)
    if with_cnt:
        k = pl.kernel(body, out_type=out_type, mesh=mesh,
                      scratch_types=scratch, compiler_params=params)
        return k(table, edge3, zeros_c, zeros_w, ones_w)
    k = pl.kernel(body, out_type=out_type[0], mesh=mesh,
                  scratch_types=scratch, compiler_params=params)
    return k(table, edge3, zeros_c)


def _tc_pre(x, W3l, W3r):
    BN = 640
    D = x.shape[1]
    H = W3l.shape[1]

    def body(x_ref, wl_ref, wr_ref, p_ref, xr_ref):
        xb = x_ref[...]
        p_ref[...] = jnp.dot(xb, wl_ref[...], preferred_element_type=jnp.float32)
        xr_ref[...] = jnp.dot(xb, wr_ref[...], preferred_element_type=jnp.float32)

    return pl.pallas_call(
        body,
        grid=(NPAD // BN,),
        in_specs=[
            pl.BlockSpec((BN, D), lambda i: (i, 0)),
            pl.BlockSpec((D, H), lambda i: (0, 0)),
            pl.BlockSpec((D, H), lambda i: (0, 0)),
        ],
        out_specs=[pl.BlockSpec((BN, H), lambda i: (i, 0))] * 2,
        out_shape=[jax.ShapeDtypeStruct((NPAD, H), jnp.float32)] * 2,
    )(x, W3l, W3r)


def _sc_layer4_fused(acc1, cnt, xrp, b3, edge3):
    C = 32
    mesh = plsc.VectorSubcoreMesh(core_axis_name="c", subcore_axis_name="s")
    out_type = [jax.ShapeDtypeStruct((NPAD, C), jnp.float32),
                jax.ShapeDtypeStruct((NC, NPAD, C), jnp.float32)]
    scratch = [
        pltpu.VMEM((NCHUNK, CH), jnp.int32),
        pltpu.VMEM((NCHUNK, CH), jnp.int32),
        [pltpu.VMEM((CH, C), jnp.float32) for _ in range(NBUF)],
        [pltpu.SemaphoreType.DMA for _ in range(NBUF)],
        pltpu.VMEM_SHARED((NPAD, C), jnp.float32),
        pltpu.VMEM((RPT, C), jnp.float32),
        pltpu.VMEM((RPT, C), jnp.float32),
        pltpu.VMEM((RPT, C), jnp.float32),
        pltpu.VMEM((RPT, CNTW), jnp.float32),
        pltpu.VMEM((RPT, CNTW), jnp.float32),
        pltpu.VMEM((RPT,), jnp.float32),
        pltpu.VMEM((C,), jnp.float32),
    ]
    zeros_c = jnp.zeros((NPAD, C), jnp.float32)

    def body(a1_h, cnt_h, xr_h, b3_h, e_h, zc_h, h_o, acc_o,
             sidx, didx, rows, sems, acc_sh, a0v, a1v, xrv, c0v, c1v,
             rdv, b3v):
        c = lax.axis_index("c")
        s = lax.axis_index("s")
        wid = c * NS + s
        r0 = s * RPT
        pltpu.sync_copy(zc_h.at[pl.ds(r0, RPT)], acc_sh.at[pl.ds(r0, RPT)])
        pltpu.sync_copy(e_h.at[0, wid], sidx)
        pltpu.sync_copy(e_h.at[1, wid], didx)
        pltpu.sync_copy(a1_h.at[0, pl.ds(r0, RPT)], a0v)
        pltpu.sync_copy(a1_h.at[1, pl.ds(r0, RPT)], a1v)
        pltpu.sync_copy(cnt_h.at[0, pl.ds(r0, RPT)], c0v)
        pltpu.sync_copy(cnt_h.at[1, pl.ds(r0, RPT)], c1v)
        pltpu.sync_copy(xr_h.at[pl.ds(r0, RPT)], xrv)
        pltpu.sync_copy(b3_h, b3v)

        @pl.loop(0, RPT, step=16)
        def deg16(g):
            ridx = g + lax.iota(jnp.int32, 16)
            z16 = jnp.zeros((16,), jnp.int32)
            d0 = plsc.load_gather(c0v, [ridx, z16])
            d1 = plsc.load_gather(c1v, [ridx, z16])
            rdv[pl.ds(g, 16)] = 1.0 / jnp.maximum(d0 + d1, 1.0)

        @pl.loop(0, RPT)
        def hrow(r):
            rd = plsc.load_gather(rdv, [jnp.full((16,), r, jnp.int32)])
            for half in range(2):
                cs = pl.ds(half * 16, 16)
                v = ((a0v[r, cs] + a1v[r, cs]) * rd + b3v[cs] + xrv[r, cs])
                xrv[r, cs] = jnp.maximum(v, 0.0)

        pltpu.sync_copy(xrv, h_o.at[pl.ds(r0, RPT)])
        plsc.subcore_barrier()

        @pl.loop(0, NCHUNK, step=NBUF)
        def group(g):
            ds = [pltpu.async_copy(h_o.at[sidx.at[g + b]], rows[b], sems[b])
                  for b in range(NBUF)]
            for b in range(NBUF):
                ds[b].wait()
                pltpu.sync_copy(rows[b], acc_sh.at[didx.at[g + b]], add=True)

        plsc.subcore_barrier()
        pltpu.sync_copy(acc_sh.at[pl.ds(r0, RPT)], acc_o.at[c, pl.ds(r0, RPT)])

    k = pl.kernel(body, out_type=out_type, mesh=mesh, scratch_types=scratch,
                  compiler_params=pltpu.CompilerParams(
                      use_tc_tiling_on_sc=False, needs_layout_passes=False))
    return k(acc1, cnt, xrp, b3, edge3, zeros_c)


def _tc_mid(acc, cnt, xr, b3):
    BN = 1000
    H = xr.shape[1]

    def body(a_ref, c_ref, xr_ref, b_ref, h_ref):
        a = a_ref[...]
        cn = c_ref[...]
        ssum = a[0] + a[1]
        deg = cn[0, :, :1] + cn[1, :, :1]
        h_ref[...] = jnp.maximum(
            ssum / jnp.maximum(deg, 1.0) + b_ref[...] + xr_ref[...], 0.0)

    return pl.pallas_call(
        body,
        grid=(N // BN,),
        in_specs=[
            pl.BlockSpec((NC, BN, H), lambda i: (0, i, 0)),
            pl.BlockSpec((NC, BN, CNTW), lambda i: (0, i, 0)),
            pl.BlockSpec((BN, H), lambda i: (i, 0)),
            pl.BlockSpec((1, H), lambda i: (0, 0)),
        ],
        out_specs=pl.BlockSpec((BN, H), lambda i: (i, 0)),
        out_shape=jax.ShapeDtypeStruct((N, H), jnp.float32),
    )(acc, cnt, xr, b3)


def _tc_out(acc, cnt, h, W4l, W4r, b4):
    BN = 1000
    H = h.shape[1]
    O = W4l.shape[1]

    def body(a_ref, c_ref, h_ref, wl_ref, wr_ref, b_ref, o_ref):
        a = a_ref[...]
        cn = c_ref[...]
        deg = cn[0, :, :1] + cn[1, :, :1]
        mean = (a[0] + a[1]) / jnp.maximum(deg, 1.0)
        o = (jnp.dot(mean, wl_ref[...], preferred_element_type=jnp.float32)
             + b_ref[...]
             + jnp.dot(h_ref[...], wr_ref[...],
                       preferred_element_type=jnp.float32))
        m = jnp.max(o, axis=1, keepdims=True)
        eo = jnp.exp(o - m)
        o_ref[...] = o - m - jnp.log(jnp.sum(eo, axis=1, keepdims=True))

    return pl.pallas_call(
        body,
        grid=(N // BN,),
        in_specs=[
            pl.BlockSpec((NC, BN, H), lambda i: (0, i, 0)),
            pl.BlockSpec((NC, BN, CNTW), lambda i: (0, i, 0)),
            pl.BlockSpec((BN, H), lambda i: (i, 0)),
            pl.BlockSpec((H, O), lambda i: (0, 0)),
            pl.BlockSpec((H, O), lambda i: (0, 0)),
            pl.BlockSpec((1, O), lambda i: (0, 0)),
        ],
        out_specs=pl.BlockSpec((BN, O), lambda i: (i, 0)),
        out_shape=jax.ShapeDtypeStruct((N, O), jnp.float32),
    )(acc, cnt, h, W4l, W4r, b4)


def kernel(x, edge_index, W1l, b1, W1r, W2l, b2, W2r, W3l, b3, W3r,
           W4l, b4, W4r):
    edge3 = edge_index.reshape(2, NW, NCHUNK, CH)
    p, xr = _tc_pre(x, W3l, W3r)
    acc1, cnt = _seg_sum_sc(p, edge3, with_cnt=True)
    h, acc2 = _sc_layer4_fused(acc1, cnt, xr, b3, edge3)
    return _tc_out(acc2, cnt, h, W4l, W4r, b4.reshape(1, -1))


# speedup vs baseline: 16.7542x; 1.0281x over previous
import functools

import jax
import jax.numpy as jnp
from jax import lax
from jax.experimental import pallas as pl
from jax.experimental.pallas import tpu as pltpu
from jax.experimental.pallas import tpu_sc as plsc

N = 10000
E = 320000
NC = 2
NS = 16
NW = NC * NS
EPW = E // NW
CH = 125
NCHUNK = EPW // CH
NBUF = 4
NPAD = 10240
RPT = NPAD // NS
CNTW = 8


def _seg_sum_sc(table, edge3, with_cnt):
    C = table.shape[1]
    NB1 = 8
    mesh = plsc.VectorSubcoreMesh(core_axis_name="c", subcore_axis_name="s")
    out_type = [jax.ShapeDtypeStruct((NC, NPAD, C), jnp.float32)]
    scratch = [
        pltpu.VMEM((NCHUNK, CH), jnp.int32),
        pltpu.VMEM((NCHUNK, CH), jnp.int32),
        [pltpu.VMEM((CH, C), jnp.float32) for _ in range(NB1)],
        [pltpu.SemaphoreType.DMA for _ in range(NB1)],
        [pltpu.SemaphoreType.DMA for _ in range(NB1)],
        pltpu.SemaphoreType.DMA,
        pltpu.VMEM_SHARED((NPAD, C), jnp.float32),
    ]
    if with_cnt:
        out_type.append(jax.ShapeDtypeStruct((NC, NPAD, CNTW), jnp.float32))
        scratch += [
            pltpu.VMEM((CH, CNTW), jnp.float32),
            pltpu.VMEM_SHARED((NPAD, CNTW), jnp.float32),
        ]

    zeros_c = jnp.zeros((NPAD, C), jnp.float32)
    if with_cnt:
        zeros_w = jnp.zeros((NPAD, CNTW), jnp.float32)
        ones_w = jnp.ones((CH, CNTW), jnp.float32)

    def body(*refs):
        if with_cnt:
            (tbl, e_h, zc_h, zw_h, ones_h, acc_o, cnt_o,
             sidx, didx, rows, sems, ssems, csem, acc_sh,
             ones_v, cnt_sh) = refs
        else:
            (tbl, e_h, zc_h, acc_o,
             sidx, didx, rows, sems, ssems, csem, acc_sh) = refs
        c = lax.axis_index("c")
        s = lax.axis_index("s")
        wid = c * NS + s
        r0 = s * RPT
        pltpu.sync_copy(zc_h.at[pl.ds(r0, RPT)], acc_sh.at[pl.ds(r0, RPT)])
        pltpu.sync_copy(e_h.at[0, wid], sidx)
        pltpu.sync_copy(e_h.at[1, wid], didx)
        if with_cnt:
            pltpu.sync_copy(zw_h.at[pl.ds(r0, RPT)], cnt_sh.at[pl.ds(r0, RPT)])
            pltpu.sync_copy(ones_h, ones_v)
        plsc.subcore_barrier()

        @pl.loop(0, NCHUNK, step=NB1)
        def group(g):
            ds = [pltpu.async_copy(tbl.at[sidx.at[g + b]], rows[b], sems[b])
                  for b in range(NB1)]
            for b in range(NB1):
                ds[b].wait()
                pltpu.sync_copy(rows[b], acc_sh.at[didx.at[g + b]], add=True)
                if with_cnt:
                    pltpu.sync_copy(ones_v, cnt_sh.at[didx.at[g + b]],
                                    add=True)

        plsc.subcore_barrier()
        pltpu.sync_copy(acc_sh.at[pl.ds(r0, RPT)], acc_o.at[c, pl.ds(r0, RPT)])
        if with_cnt:
            pltpu.sync_copy(cnt_sh.at[pl.ds(r0, RPT)],
                            cnt_o.at[c, pl.ds(r0, RPT)])

    params = pltpu.CompilerParams(use_tc_tiling_on_sc=False)
    if with_cnt:
        k = pl.kernel(body, out_type=out_type, mesh=mesh,
                      scratch_types=scratch, compiler_params=params)
        return k(table, edge3, zeros_c, zeros_w, ones_w)
    k = pl.kernel(body, out_type=out_type[0], mesh=mesh,
                  scratch_types=scratch, compiler_params=params)
    return k(table, edge3, zeros_c)


def _tc_pre(x, W3l, W3r):
    BN = 640
    D = x.shape[1]
    H = W3l.shape[1]

    def body(x_ref, wl_ref, wr_ref, p_ref, xr_ref):
        xb = x_ref[...]
        p_ref[...] = jnp.dot(xb, wl_ref[...], preferred_element_type=jnp.float32)
        xr_ref[...] = jnp.dot(xb, wr_ref[...], preferred_element_type=jnp.float32)

    return pl.pallas_call(
        body,
        grid=(NPAD // BN,),
        in_specs=[
            pl.BlockSpec((BN, D), lambda i: (i, 0)),
            pl.BlockSpec((D, H), lambda i: (0, 0)),
            pl.BlockSpec((D, H), lambda i: (0, 0)),
        ],
        out_specs=[pl.BlockSpec((BN, H), lambda i: (i, 0))] * 2,
        out_shape=[jax.ShapeDtypeStruct((NPAD, H), jnp.float32)] * 2,
    )(x, W3l, W3r)


def _sc_layer4_fused(acc1, cnt, xrp, b3, edge3):
    C = 32
    mesh = plsc.VectorSubcoreMesh(core_axis_name="c", subcore_axis_name="s")
    out_type = [jax.ShapeDtypeStruct((NPAD, C), jnp.float32),
                jax.ShapeDtypeStruct((NC, NPAD, C), jnp.float32)]
    scratch = [
        pltpu.VMEM((NCHUNK, CH), jnp.int32),
        pltpu.VMEM((NCHUNK, CH), jnp.int32),
        [pltpu.VMEM((CH, C), jnp.float32) for _ in range(NBUF)],
        [pltpu.SemaphoreType.DMA for _ in range(NBUF)],
        [pltpu.SemaphoreType.DMA for _ in range(NBUF)],
        pltpu.VMEM_SHARED((NPAD, C), jnp.float32),
        pltpu.VMEM((RPT, C), jnp.float32),
        pltpu.VMEM((RPT, C), jnp.float32),
        pltpu.VMEM((RPT, C), jnp.float32),
        pltpu.VMEM((RPT, CNTW), jnp.float32),
        pltpu.VMEM((RPT, CNTW), jnp.float32),
        pltpu.VMEM((RPT,), jnp.float32),
        pltpu.VMEM((C,), jnp.float32),
    ]
    zeros_c = jnp.zeros((NPAD, C), jnp.float32)

    def body(a1_h, cnt_h, xr_h, b3_h, e_h, zc_h, h_o, acc_o,
             sidx, didx, rows, sems, ssems, acc_sh, a0v, a1v, xrv, c0v, c1v,
             rdv, b3v):
        c = lax.axis_index("c")
        s = lax.axis_index("s")
        wid = c * NS + s
        r0 = s * RPT
        pltpu.sync_copy(zc_h.at[pl.ds(r0, RPT)], acc_sh.at[pl.ds(r0, RPT)])
        pltpu.sync_copy(e_h.at[0, wid], sidx)
        pltpu.sync_copy(e_h.at[1, wid], didx)
        pltpu.sync_copy(a1_h.at[0, pl.ds(r0, RPT)], a0v)
        pltpu.sync_copy(a1_h.at[1, pl.ds(r0, RPT)], a1v)
        pltpu.sync_copy(cnt_h.at[0, pl.ds(r0, RPT)], c0v)
        pltpu.sync_copy(cnt_h.at[1, pl.ds(r0, RPT)], c1v)
        pltpu.sync_copy(xr_h.at[pl.ds(r0, RPT)], xrv)
        pltpu.sync_copy(b3_h, b3v)

        @pl.loop(0, RPT, step=16)
        def deg16(g):
            ridx = g + lax.iota(jnp.int32, 16)
            z16 = jnp.zeros((16,), jnp.int32)
            d0 = plsc.load_gather(c0v, [ridx, z16])
            d1 = plsc.load_gather(c1v, [ridx, z16])
            rdv[pl.ds(g, 16)] = 1.0 / jnp.maximum(d0 + d1, 1.0)

        @pl.loop(0, RPT, unroll=4)
        def hrow(r):
            rd = plsc.load_gather(rdv, [jnp.full((16,), r, jnp.int32)])
            for half in range(2):
                cs = pl.ds(half * 16, 16)
                v = ((a0v[r, cs] + a1v[r, cs]) * rd + b3v[cs] + xrv[r, cs])
                xrv[r, cs] = jnp.maximum(v, 0.0)

        pltpu.sync_copy(xrv, h_o.at[pl.ds(r0, RPT)])
        plsc.subcore_barrier()

        @pl.loop(0, NCHUNK, step=NBUF)
        def group(g):
            ds = [pltpu.async_copy(h_o.at[sidx.at[g + b]], rows[b], sems[b])
                  for b in range(NBUF)]
            for b in range(NBUF):
                ds[b].wait()
                pltpu.sync_copy(rows[b], acc_sh.at[didx.at[g + b]], add=True)

        plsc.subcore_barrier()
        pltpu.sync_copy(acc_sh.at[pl.ds(r0, RPT)], acc_o.at[c, pl.ds(r0, RPT)])

    k = pl.kernel(body, out_type=out_type, mesh=mesh, scratch_types=scratch,
                  compiler_params=pltpu.CompilerParams(
                      use_tc_tiling_on_sc=False, needs_layout_passes=False))
    return k(acc1, cnt, xrp, b3, edge3, zeros_c)


def _tc_mid(acc, cnt, xr, b3):
    BN = 1000
    H = xr.shape[1]

    def body(a_ref, c_ref, xr_ref, b_ref, h_ref):
        a = a_ref[...]
        cn = c_ref[...]
        ssum = a[0] + a[1]
        deg = cn[0, :, :1] + cn[1, :, :1]
        h_ref[...] = jnp.maximum(
            ssum / jnp.maximum(deg, 1.0) + b_ref[...] + xr_ref[...], 0.0)

    return pl.pallas_call(
        body,
        grid=(N // BN,),
        in_specs=[
            pl.BlockSpec((NC, BN, H), lambda i: (0, i, 0)),
            pl.BlockSpec((NC, BN, CNTW), lambda i: (0, i, 0)),
            pl.BlockSpec((BN, H), lambda i: (i, 0)),
            pl.BlockSpec((1, H), lambda i: (0, 0)),
        ],
        out_specs=pl.BlockSpec((BN, H), lambda i: (i, 0)),
        out_shape=jax.ShapeDtypeStruct((N, H), jnp.float32),
    )(acc, cnt, xr, b3)


def _tc_out(acc, cnt, h, W4l, W4r, b4):
    BN = 1000
    H = h.shape[1]
    O = W4l.shape[1]

    def body(a_ref, c_ref, h_ref, wl_ref, wr_ref, b_ref, o_ref):
        a = a_ref[...]
        cn = c_ref[...]
        deg = cn[0, :, :1] + cn[1, :, :1]
        mean = (a[0] + a[1]) / jnp.maximum(deg, 1.0)
        o = (jnp.dot(mean, wl_ref[...], preferred_element_type=jnp.float32)
             + b_ref[...]
             + jnp.dot(h_ref[...], wr_ref[...],
                       preferred_element_type=jnp.float32))
        m = jnp.max(o, axis=1, keepdims=True)
        eo = jnp.exp(o - m)
        o_ref[...] = o - m - jnp.log(jnp.sum(eo, axis=1, keepdims=True))

    return pl.pallas_call(
        body,
        grid=(N // BN,),
        in_specs=[
            pl.BlockSpec((NC, BN, H), lambda i: (0, i, 0)),
            pl.BlockSpec((NC, BN, CNTW), lambda i: (0, i, 0)),
            pl.BlockSpec((BN, H), lambda i: (i, 0)),
            pl.BlockSpec((H, O), lambda i: (0, 0)),
            pl.BlockSpec((H, O), lambda i: (0, 0)),
            pl.BlockSpec((1, O), lambda i: (0, 0)),
        ],
        out_specs=pl.BlockSpec((BN, O), lambda i: (i, 0)),
        out_shape=jax.ShapeDtypeStruct((N, O), jnp.float32),
    )(acc, cnt, h, W4l, W4r, b4)


def kernel(x, edge_index, W1l, b1, W1r, W2l, b2, W2r, W3l, b3, W3r,
           W4l, b4, W4r):
    edge3 = edge_index.reshape(2, NW, NCHUNK, CH)
    p, xr = _tc_pre(x, W3l, W3r)
    acc1, cnt = _seg_sum_sc(p, edge3, with_cnt=True)
    h, acc2 = _sc_layer4_fused(acc1, cnt, xr, b3, edge3)
    return _tc_out(acc2, cnt, h, W4l, W4r, b4.reshape(1, -1))


# speedup vs baseline: 17.2111x; 1.0273x over previous
import functools

import jax
import jax.numpy as jnp
from jax import lax
from jax.experimental import pallas as pl
from jax.experimental.pallas import tpu as pltpu
from jax.experimental.pallas import tpu_sc as plsc

N = 10000
E = 320000
NC = 2
NS = 16
NW = NC * NS
EPW = E // NW
CH = 125
NCHUNK = EPW // CH
NBUF = 4
NPAD = 10240
RPT = NPAD // NS
CNTW = 8


def _seg_sum_sc(table, edge3):
    C = table.shape[1]
    NB1 = 8
    mesh = plsc.VectorSubcoreMesh(core_axis_name="c", subcore_axis_name="s")
    out_type = jax.ShapeDtypeStruct((NC, NPAD, C), jnp.float32)
    scratch = [
        pltpu.VMEM((NCHUNK, CH), jnp.int32),
        pltpu.VMEM((NCHUNK, CH), jnp.int32),
        [pltpu.VMEM((CH, C), jnp.float32) for _ in range(NB1)],
        [pltpu.SemaphoreType.DMA for _ in range(NB1)],
        pltpu.VMEM_SHARED((NPAD, C), jnp.float32),
    ]
    zeros_c = jnp.zeros((NPAD, C), jnp.float32)

    def body(tbl, e_h, zc_h, acc_o, sidx, didx, rows, sems, acc_sh):
        c = lax.axis_index("c")
        s = lax.axis_index("s")
        wid = c * NS + s
        r0 = s * RPT
        pltpu.sync_copy(zc_h.at[pl.ds(r0, RPT)], acc_sh.at[pl.ds(r0, RPT)])
        pltpu.sync_copy(e_h.at[0, wid], sidx)
        pltpu.sync_copy(e_h.at[1, wid], didx)
        plsc.subcore_barrier()

        @pl.loop(0, NCHUNK, step=NB1)
        def group(g):
            ds = [pltpu.async_copy(tbl.at[sidx.at[g + b]], rows[b], sems[b])
                  for b in range(NB1)]
            for b in range(NB1):
                ds[b].wait()
                pltpu.sync_copy(rows[b], acc_sh.at[didx.at[g + b]], add=True)

        plsc.subcore_barrier()
        pltpu.sync_copy(acc_sh.at[pl.ds(r0, RPT)], acc_o.at[c, pl.ds(r0, RPT)])

    k = pl.kernel(body, out_type=out_type, mesh=mesh,
                  scratch_types=scratch,
                  compiler_params=pltpu.CompilerParams(
                      use_tc_tiling_on_sc=False))
    return k(table, edge3, zeros_c)


def _tc_pre(x, W3l, W3r):
    BN = 640
    D = x.shape[1]
    H = W3l.shape[1]

    def body(x_ref, wl_ref, wr_ref, p_ref, xr_ref):
        xb = x_ref[...]
        pb = jnp.dot(xb, wl_ref[...], preferred_element_type=jnp.float32)
        p_ref[...] = jnp.concatenate(
            [pb, jnp.ones((BN, CNTW), jnp.float32)], axis=1)
        xr_ref[...] = jnp.dot(xb, wr_ref[...], preferred_element_type=jnp.float32)

    return pl.pallas_call(
        body,
        grid=(NPAD // BN,),
        in_specs=[
            pl.BlockSpec((BN, D), lambda i: (i, 0)),
            pl.BlockSpec((D, H), lambda i: (0, 0)),
            pl.BlockSpec((D, H), lambda i: (0, 0)),
        ],
        out_specs=[pl.BlockSpec((BN, H + CNTW), lambda i: (i, 0)),
                   pl.BlockSpec((BN, H), lambda i: (i, 0))],
        out_shape=[jax.ShapeDtypeStruct((NPAD, H + CNTW), jnp.float32),
                   jax.ShapeDtypeStruct((NPAD, H), jnp.float32)],
    )(x, W3l, W3r)


def _sc_layer4_fused(acc1, xrp, b3, edge3):
    C = 32
    CW = acc1.shape[2]
    mesh = plsc.VectorSubcoreMesh(core_axis_name="c", subcore_axis_name="s")
    out_type = [jax.ShapeDtypeStruct((NPAD, C), jnp.float32),
                jax.ShapeDtypeStruct((NC, NPAD, C), jnp.float32)]
    scratch = [
        pltpu.VMEM((NCHUNK, CH), jnp.int32),
        pltpu.VMEM((NCHUNK, CH), jnp.int32),
        [pltpu.VMEM((CH, C), jnp.float32) for _ in range(NBUF)],
        [pltpu.SemaphoreType.DMA for _ in range(NBUF)],
        pltpu.VMEM_SHARED((NPAD, C), jnp.float32),
        pltpu.VMEM((RPT, CW), jnp.float32),
        pltpu.VMEM((RPT, CW), jnp.float32),
        pltpu.VMEM((RPT, C), jnp.float32),
        pltpu.VMEM((RPT,), jnp.float32),
        pltpu.VMEM((C,), jnp.float32),
    ]
    zeros_c = jnp.zeros((NPAD, C), jnp.float32)

    def body(a1_h, xr_h, b3_h, e_h, zc_h, h_o, acc_o,
             sidx, didx, rows, sems, acc_sh, a0v, a1v, xrv, rdv, b3v):
        c = lax.axis_index("c")
        s = lax.axis_index("s")
        wid = c * NS + s
        r0 = s * RPT
        pltpu.sync_copy(zc_h.at[pl.ds(r0, RPT)], acc_sh.at[pl.ds(r0, RPT)])
        pltpu.sync_copy(e_h.at[0, wid], sidx)
        pltpu.sync_copy(e_h.at[1, wid], didx)
        pltpu.sync_copy(a1_h.at[0, pl.ds(r0, RPT)], a0v)
        pltpu.sync_copy(a1_h.at[1, pl.ds(r0, RPT)], a1v)
        pltpu.sync_copy(xr_h.at[pl.ds(r0, RPT)], xrv)
        pltpu.sync_copy(b3_h, b3v)

        @pl.loop(0, RPT, step=16)
        def deg16(g):
            ridx = g + lax.iota(jnp.int32, 16)
            c32 = jnp.full((16,), C, jnp.int32)
            d0 = plsc.load_gather(a0v, [ridx, c32])
            d1 = plsc.load_gather(a1v, [ridx, c32])
            rdv[pl.ds(g, 16)] = 1.0 / jnp.maximum(d0 + d1, 1.0)

        @pl.loop(0, RPT, unroll=4)
        def hrow(r):
            rd = plsc.load_gather(rdv, [jnp.full((16,), r, jnp.int32)])
            for half in range(2):
                cs = pl.ds(half * 16, 16)
                v = ((a0v[r, cs] + a1v[r, cs]) * rd + b3v[cs] + xrv[r, cs])
                xrv[r, cs] = jnp.maximum(v, 0.0)

        pltpu.sync_copy(xrv, h_o.at[pl.ds(r0, RPT)])
        plsc.subcore_barrier()

        @pl.loop(0, NCHUNK, step=NBUF)
        def group(g):
            ds = [pltpu.async_copy(h_o.at[sidx.at[g + b]], rows[b], sems[b])
                  for b in range(NBUF)]
            for b in range(NBUF):
                ds[b].wait()
                pltpu.sync_copy(rows[b], acc_sh.at[didx.at[g + b]], add=True)

        plsc.subcore_barrier()
        pltpu.sync_copy(acc_sh.at[pl.ds(r0, RPT)], acc_o.at[c, pl.ds(r0, RPT)])

    k = pl.kernel(body, out_type=out_type, mesh=mesh, scratch_types=scratch,
                  compiler_params=pltpu.CompilerParams(
                      use_tc_tiling_on_sc=False, needs_layout_passes=False))
    return k(acc1, xrp, b3, edge3, zeros_c)


def _tc_out(acc2, acc1, h, W4l, W4r, b4):
    BN = 1000
    H = h.shape[1]
    CW = acc1.shape[2]
    O = W4l.shape[1]

    def body(a_ref, a1_ref, h_ref, wl_ref, wr_ref, b_ref, o_ref):
        a = a_ref[...]
        a1 = a1_ref[...]
        deg = a1[0, :, H:H + 1] + a1[1, :, H:H + 1]
        mean = (a[0] + a[1]) / jnp.maximum(deg, 1.0)
        o = (jnp.dot(mean, wl_ref[...], preferred_element_type=jnp.float32)
             + b_ref[...]
             + jnp.dot(h_ref[...], wr_ref[...],
                       preferred_element_type=jnp.float32))
        m = jnp.max(o, axis=1, keepdims=True)
        eo = jnp.exp(o - m)
        o_ref[...] = o - m - jnp.log(jnp.sum(eo, axis=1, keepdims=True))

    return pl.pallas_call(
        body,
        grid=(N // BN,),
        in_specs=[
            pl.BlockSpec((NC, BN, H), lambda i: (0, i, 0)),
            pl.BlockSpec((NC, BN, CW), lambda i: (0, i, 0)),
            pl.BlockSpec((BN, H), lambda i: (i, 0)),
            pl.BlockSpec((H, O), lambda i: (0, 0)),
            pl.BlockSpec((H, O), lambda i: (0, 0)),
            pl.BlockSpec((1, O), lambda i: (0, 0)),
        ],
        out_specs=pl.BlockSpec((BN, O), lambda i: (i, 0)),
        out_shape=jax.ShapeDtypeStruct((N, O), jnp.float32),
    )(acc2, acc1, h, W4l, W4r, b4)


def kernel(x, edge_index, W1l, b1, W1r, W2l, b2, W2r, W3l, b3, W3r,
           W4l, b4, W4r):
    edge3 = edge_index.reshape(2, NW, NCHUNK, CH)
    p, xr = _tc_pre(x, W3l, W3r)
    acc1 = _seg_sum_sc(p, edge3)
    h, acc2 = _sc_layer4_fused(acc1, xr, b3, edge3)
    return _tc_out(acc2, acc1, h, W4l, W4r, b4.reshape(1, -1))


# speedup vs baseline: 18.0996x; 1.0516x over previous
import functools

import jax
import jax.numpy as jnp
from jax import lax
from jax.experimental import pallas as pl
from jax.experimental.pallas import tpu as pltpu
from jax.experimental.pallas import tpu_sc as plsc

N = 10000
E = 320000
NC = 2
NS = 16
NW = NC * NS
EPW = E // NW
CH = 125
NCHUNK = EPW // CH
NBUF = 4
NPAD = 10240
RPT = NPAD // NS
CNTW = 8


def _seg_sum_sc(table, edge3):
    C = table.shape[1]
    NB1 = 8
    mesh = plsc.VectorSubcoreMesh(core_axis_name="c", subcore_axis_name="s")
    out_type = jax.ShapeDtypeStruct((NC, NPAD, C), jnp.float32)
    scratch = [
        pltpu.VMEM((NCHUNK, CH), jnp.int32),
        pltpu.VMEM((NCHUNK, CH), jnp.int32),
        [pltpu.VMEM((CH, C), jnp.float32) for _ in range(NB1)],
        [pltpu.SemaphoreType.DMA for _ in range(NB1)],
        pltpu.VMEM_SHARED((NPAD, C), jnp.float32),
    ]
    zeros_c = jnp.zeros((NPAD, C), jnp.float32)

    def body(tbl, e_h, zc_h, acc_o, sidx, didx, rows, sems, acc_sh):
        c = lax.axis_index("c")
        s = lax.axis_index("s")
        wid = c * NS + s
        r0 = s * RPT
        pltpu.sync_copy(zc_h.at[pl.ds(r0, RPT)], acc_sh.at[pl.ds(r0, RPT)])
        pltpu.sync_copy(e_h.at[0, wid], sidx)
        pltpu.sync_copy(e_h.at[1, wid], didx)
        plsc.subcore_barrier()

        @pl.loop(0, NCHUNK, step=NB1)
        def group(g):
            ds = [pltpu.async_copy(tbl.at[sidx.at[g + b]], rows[b], sems[b])
                  for b in range(NB1)]
            for b in range(NB1):
                ds[b].wait()
                pltpu.sync_copy(rows[b], acc_sh.at[didx.at[g + b]], add=True)

        plsc.subcore_barrier()
        pltpu.sync_copy(acc_sh.at[pl.ds(r0, RPT)], acc_o.at[c, pl.ds(r0, RPT)])

    k = pl.kernel(body, out_type=out_type, mesh=mesh,
                  scratch_types=scratch,
                  compiler_params=pltpu.CompilerParams(
                      use_tc_tiling_on_sc=False))
    return k(table, edge3, zeros_c)


def _tc_pre(x, W3l, W3r):
    BN = 1280
    D = x.shape[1]
    H = W3l.shape[1]

    def body(x_ref, wl_ref, wr_ref, p_ref, xr_ref):
        xb = x_ref[...]
        pb = jnp.dot(xb, wl_ref[...], preferred_element_type=jnp.float32)
        p_ref[...] = jnp.concatenate(
            [pb, jnp.ones((BN, CNTW), jnp.float32)], axis=1)
        xr_ref[...] = jnp.dot(xb, wr_ref[...], preferred_element_type=jnp.float32)

    return pl.pallas_call(
        body,
        grid=(NPAD // BN,),
        in_specs=[
            pl.BlockSpec((BN, D), lambda i: (i, 0)),
            pl.BlockSpec((D, H), lambda i: (0, 0)),
            pl.BlockSpec((D, H), lambda i: (0, 0)),
        ],
        out_specs=[pl.BlockSpec((BN, H + CNTW), lambda i: (i, 0)),
                   pl.BlockSpec((BN, H), lambda i: (i, 0))],
        out_shape=[jax.ShapeDtypeStruct((NPAD, H + CNTW), jnp.float32),
                   jax.ShapeDtypeStruct((NPAD, H), jnp.float32)],
    )(x, W3l, W3r)


def _sc_layer4_fused(acc1, xrp, b3, edge3):
    C = 32
    CW = acc1.shape[2]
    mesh = plsc.VectorSubcoreMesh(core_axis_name="c", subcore_axis_name="s")
    out_type = [jax.ShapeDtypeStruct((NPAD, C), jnp.float32),
                jax.ShapeDtypeStruct((NC, NPAD, C), jnp.float32)]
    scratch = [
        pltpu.VMEM((NCHUNK, CH), jnp.int32),
        pltpu.VMEM((NCHUNK, CH), jnp.int32),
        [pltpu.VMEM((CH, C), jnp.float32) for _ in range(NBUF)],
        [pltpu.SemaphoreType.DMA for _ in range(NBUF)],
        pltpu.VMEM_SHARED((NPAD, C), jnp.float32),
        pltpu.VMEM((RPT, CW), jnp.float32),
        pltpu.VMEM((RPT, CW), jnp.float32),
        pltpu.VMEM((RPT, C), jnp.float32),
        pltpu.VMEM((RPT,), jnp.float32),
        pltpu.VMEM((C,), jnp.float32),
    ]
    zeros_c = jnp.zeros((NPAD, C), jnp.float32)

    def body(a1_h, xr_h, b3_h, e_h, zc_h, h_o, acc_o,
             sidx, didx, rows, sems, acc_sh, a0v, a1v, xrv, rdv, b3v):
        c = lax.axis_index("c")
        s = lax.axis_index("s")
        wid = c * NS + s
        r0 = s * RPT
        pre = [
            pltpu.async_copy(zc_h.at[pl.ds(r0, RPT)],
                             acc_sh.at[pl.ds(r0, RPT)], sems[0]),
            pltpu.async_copy(e_h.at[0, wid], sidx, sems[1]),
            pltpu.async_copy(e_h.at[1, wid], didx, sems[2]),
            pltpu.async_copy(a1_h.at[0, pl.ds(r0, RPT)], a0v, sems[3]),
        ]
        pltpu.sync_copy(a1_h.at[1, pl.ds(r0, RPT)], a1v)
        pltpu.sync_copy(xr_h.at[pl.ds(r0, RPT)], xrv)
        pltpu.sync_copy(b3_h, b3v)
        for d in pre:
            d.wait()

        @pl.loop(0, RPT, step=16)
        def deg16(g):
            ridx = g + lax.iota(jnp.int32, 16)
            c32 = jnp.full((16,), C, jnp.int32)
            d0 = plsc.load_gather(a0v, [ridx, c32])
            d1 = plsc.load_gather(a1v, [ridx, c32])
            rdv[pl.ds(g, 16)] = 1.0 / jnp.maximum(d0 + d1, 1.0)

        @pl.loop(0, RPT, unroll=4)
        def hrow(r):
            rd = plsc.load_gather(rdv, [jnp.full((16,), r, jnp.int32)])
            for half in range(2):
                cs = pl.ds(half * 16, 16)
                v = ((a0v[r, cs] + a1v[r, cs]) * rd + b3v[cs] + xrv[r, cs])
                xrv[r, cs] = jnp.maximum(v, 0.0)

        pltpu.sync_copy(xrv, h_o.at[pl.ds(r0, RPT)])
        plsc.subcore_barrier()

        @pl.loop(0, NCHUNK, step=NBUF)
        def group(g):
            ds = [pltpu.async_copy(h_o.at[sidx.at[g + b]], rows[b], sems[b])
                  for b in range(NBUF)]
            for b in range(NBUF):
                ds[b].wait()
                pltpu.sync_copy(rows[b], acc_sh.at[didx.at[g + b]], add=True)

        plsc.subcore_barrier()
        pltpu.sync_copy(acc_sh.at[pl.ds(r0, RPT)], acc_o.at[c, pl.ds(r0, RPT)])

    k = pl.kernel(body, out_type=out_type, mesh=mesh, scratch_types=scratch,
                  compiler_params=pltpu.CompilerParams(
                      use_tc_tiling_on_sc=False, needs_layout_passes=False))
    return k(acc1, xrp, b3, edge3, zeros_c)


def _tc_out(acc2, acc1, h, W4l, W4r, b4):
    BN = 2000
    H = h.shape[1]
    CW = acc1.shape[2]
    O = W4l.shape[1]

    def body(a_ref, a1_ref, h_ref, wl_ref, wr_ref, b_ref, o_ref):
        a = a_ref[...]
        a1 = a1_ref[...]
        deg = a1[0, :, H:H + 1] + a1[1, :, H:H + 1]
        mean = (a[0] + a[1]) / jnp.maximum(deg, 1.0)
        o = (jnp.dot(mean, wl_ref[...], preferred_element_type=jnp.float32)
             + b_ref[...]
             + jnp.dot(h_ref[...], wr_ref[...],
                       preferred_element_type=jnp.float32))
        m = jnp.max(o, axis=1, keepdims=True)
        eo = jnp.exp(o - m)
        o_ref[...] = o - m - jnp.log(jnp.sum(eo, axis=1, keepdims=True))

    return pl.pallas_call(
        body,
        grid=(N // BN,),
        in_specs=[
            pl.BlockSpec((NC, BN, H), lambda i: (0, i, 0)),
            pl.BlockSpec((NC, BN, CW), lambda i: (0, i, 0)),
            pl.BlockSpec((BN, H), lambda i: (i, 0)),
            pl.BlockSpec((H, O), lambda i: (0, 0)),
            pl.BlockSpec((H, O), lambda i: (0, 0)),
            pl.BlockSpec((1, O), lambda i: (0, 0)),
        ],
        out_specs=pl.BlockSpec((BN, O), lambda i: (i, 0)),
        out_shape=jax.ShapeDtypeStruct((N, O), jnp.float32),
    )(acc2, acc1, h, W4l, W4r, b4)


def kernel(x, edge_index, W1l, b1, W1r, W2l, b2, W2r, W3l, b3, W3r,
           W4l, b4, W4r):
    edge3 = edge_index.reshape(2, NW, NCHUNK, CH)
    p, xr = _tc_pre(x, W3l, W3r)
    acc1 = _seg_sum_sc(p, edge3)
    h, acc2 = _sc_layer4_fused(acc1, xr, b3, edge3)
    return _tc_out(acc2, acc1, h, W4l, W4r, b4.reshape(1, -1))
